# baseline (speedup 1.0000x reference)
import functools

import jax
import jax.numpy as jnp
from jax import lax
from jax.experimental import pallas as pl
from jax.experimental.pallas import tpu as pltpu
from jax.experimental.pallas import tpu_sc as plsc

BATCH = 16384
EMBED_DIM = 32
NUM_CORES = 2
NUM_SUBCORES = 16
NUM_WORKERS = NUM_CORES * NUM_SUBCORES
B_PER_W = BATCH // NUM_WORKERS
CHUNK = 128
N_CHUNKS = B_PER_W // CHUNK

_mesh = plsc.VectorSubcoreMesh(core_axis_name="c", subcore_axis_name="s")


@functools.partial(
    pl.kernel,
    out_type=jax.ShapeDtypeStruct((EMBED_DIM, BATCH), jnp.float32),
    mesh=_mesh,
    compiler_params=pltpu.CompilerParams(use_tc_tiling_on_sc=False),
    scratch_types=[
        pltpu.VMEM((N_CHUNKS, CHUNK), jnp.int32),
        pltpu.VMEM((EMBED_DIM, B_PER_W), jnp.float32),
        pltpu.SemaphoreType.DMA,
    ],
)
def _gather_kernel(idx_hbm, table_hbm, out_hbm, idx_v, vals_v, sem):
    wid = lax.axis_index("s") * NUM_CORES + lax.axis_index("c")
    pltpu.sync_copy(idx_hbm.at[wid], idx_v)
    for dim in range(EMBED_DIM):
        copies = [
            pltpu.async_copy(
                table_hbm.at[dim].at[idx_v.at[j]],
                vals_v.at[dim, pl.ds(j * CHUNK, CHUNK)],
                sem,
            )
            for j in range(N_CHUNKS)
        ]
        for c in copies:
            c.wait()
    pltpu.sync_copy(vals_v, out_hbm.at[:, pl.ds(wid * B_PER_W, B_PER_W)])


def kernel(item_idx, table):
    idx = item_idx.astype(jnp.int32).reshape(NUM_WORKERS, N_CHUNKS, CHUNK)
    out_t = _gather_kernel(idx, table.T)
    return out_t.T


# speedup vs baseline: 4.9533x; 4.9533x over previous
import functools

import jax
import jax.numpy as jnp
from jax import lax
from jax.experimental import pallas as pl
from jax.experimental.pallas import tpu as pltpu
from jax.experimental.pallas import tpu_sc as plsc

BATCH = 16384
EMBED_DIM = 32
NUM_ROWS = 1_000_000
ROWS_PER_GATHER = 128 // EMBED_DIM
TAB_ROWS = NUM_ROWS // ROWS_PER_GATHER
NUM_CORES = 2
NUM_SUBCORES = 16
NUM_WORKERS = NUM_CORES * NUM_SUBCORES
B_PER_W = BATCH // NUM_WORKERS
CHUNK = 128
N_CHUNKS = B_PER_W // CHUNK

_mesh = plsc.VectorSubcoreMesh(core_axis_name="c", subcore_axis_name="s")


@functools.partial(
    pl.kernel,
    out_type=jax.ShapeDtypeStruct((EMBED_DIM, BATCH), jnp.float32),
    mesh=_mesh,
    compiler_params=pltpu.CompilerParams(needs_layout_passes=False),
    scratch_types=[
        pltpu.VMEM((B_PER_W,), jnp.int32),
        pltpu.VMEM((N_CHUNKS, CHUNK), jnp.int32),
        pltpu.VMEM((B_PER_W, 128), jnp.float32),
        pltpu.VMEM((EMBED_DIM, B_PER_W), jnp.float32),
        pltpu.SemaphoreType.DMA,
    ],
)
def _gather_kernel(idx_hbm, table_hbm, out_hbm, idx_v, q_v, rows_v, vals_v,
                   sem):
    wid = lax.axis_index("s") * NUM_CORES + lax.axis_index("c")
    base = wid * B_PER_W
    pltpu.sync_copy(idx_hbm.at[pl.ds(base, B_PER_W)], idx_v)

    def qbody(g, _):
        r = idx_v[pl.ds(g * 16, 16)]
        q_v.at[g // 8, pl.ds((g % 8) * 16, 16)][...] = r >> 2
        return ()

    lax.fori_loop(0, B_PER_W // 16, qbody, (), unroll=False)

    copies = [
        pltpu.async_copy(
            table_hbm.at[q_v.at[j]],
            rows_v.at[pl.ds(j * CHUNK, CHUNK)],
            sem,
        )
        for j in range(N_CHUNKS)
    ]
    for c in copies:
        c.wait()

    def ebody(g, _):
        items = lax.iota(jnp.int32, 16) + g * 16
        sub = (idx_v[pl.ds(g * 16, 16)] & 3) * EMBED_DIM

        def kbody(k, _):
            vals = plsc.load_gather(rows_v, [items, sub + k])
            plsc.store_scatter(
                vals_v, [jnp.full((16,), 0, jnp.int32) + k, items], vals
            )
            return ()

        lax.fori_loop(0, EMBED_DIM, kbody, (), unroll=False)
        return ()

    lax.fori_loop(0, B_PER_W // 16, ebody, (), unroll=False)

    pltpu.sync_copy(vals_v, out_hbm.at[:, pl.ds(base, B_PER_W)])


def kernel(item_idx, table):
    idx = item_idx.astype(jnp.int32).reshape(BATCH)
    t128 = table.reshape(TAB_ROWS, 128)
    return _gather_kernel(idx, t128).T


# speedup vs baseline: 4.9738x; 1.0041x over previous
import functools

import jax
import jax.numpy as jnp
from jax import lax
from jax.experimental import pallas as pl
from jax.experimental.pallas import tpu as pltpu
from jax.experimental.pallas import tpu_sc as plsc

BATCH = 16384
EMBED_DIM = 32
NUM_CORES = 2
NUM_SUBCORES = 16
NUM_WORKERS = NUM_CORES * NUM_SUBCORES
B_PER_W = BATCH // NUM_WORKERS
CHUNK = 128
N_CHUNKS = B_PER_W // CHUNK

_mesh = plsc.VectorSubcoreMesh(core_axis_name="c", subcore_axis_name="s")


@functools.partial(
    pl.kernel,
    out_type=jax.ShapeDtypeStruct((BATCH, EMBED_DIM), jnp.float32),
    mesh=_mesh,
    compiler_params=pltpu.CompilerParams(use_tc_tiling_on_sc=False),
    scratch_types=[
        pltpu.VMEM((N_CHUNKS, CHUNK), jnp.int32),
        pltpu.VMEM((B_PER_W, EMBED_DIM), jnp.float32),
        pltpu.SemaphoreType.DMA,
    ],
)
def _gather_kernel(idx_hbm, table_hbm, out_hbm, idx_v, rows_v, sem):
    wid = lax.axis_index("s") * NUM_CORES + lax.axis_index("c")
    pltpu.sync_copy(idx_hbm.at[wid], idx_v)
    copies = [
        pltpu.async_copy(
            table_hbm.at[idx_v.at[j]],
            rows_v.at[pl.ds(j * CHUNK, CHUNK)],
            sem,
        )
        for j in range(N_CHUNKS)
    ]
    for c in copies:
        c.wait()
    pltpu.sync_copy(rows_v, out_hbm.at[pl.ds(wid * B_PER_W, B_PER_W)])


def kernel(item_idx, table):
    idx = item_idx.astype(jnp.int32).reshape(NUM_WORKERS, N_CHUNKS, CHUNK)
    return _gather_kernel(idx, table)


# speedup vs baseline: 7.3994x; 1.4877x over previous
import functools

import jax
import jax.numpy as jnp
from jax import lax
from jax.experimental import pallas as pl
from jax.experimental.pallas import tpu as pltpu
from jax.experimental.pallas import tpu_sc as plsc

BATCH = 16384
EMBED_DIM = 32
NUM_CORES = 2
NUM_SUBCORES = 16
NUM_WORKERS = NUM_CORES * NUM_SUBCORES
B_PER_W = BATCH // NUM_WORKERS
ROUND = 64
N_ROUNDS = B_PER_W // ROUND
LANES = 16

_mesh = plsc.VectorSubcoreMesh(core_axis_name="c", subcore_axis_name="s")


@functools.partial(
    pl.kernel,
    out_type=jax.ShapeDtypeStruct((EMBED_DIM, BATCH), jnp.float32),
    mesh=_mesh,
    compiler_params=pltpu.CompilerParams(needs_layout_passes=False),
    scratch_types=[
        pltpu.VMEM((B_PER_W,), jnp.int32),
        pltpu.VMEM((ROUND * 8, EMBED_DIM), jnp.float32),
        pltpu.VMEM((EMBED_DIM, B_PER_W), jnp.float32),
        pltpu.SemaphoreType.DMA,
    ],
)
def _gather_kernel(idx_hbm, table_hbm, out_hbm, idx_v, band_v, vals_v, sem):
    wid = lax.axis_index("s") * NUM_CORES + lax.axis_index("c")
    base = wid * B_PER_W
    pltpu.sync_copy(idx_hbm.at[pl.ds(base, B_PER_W)], idx_v)

    lane_iota = lax.iota(jnp.int32, LANES)

    for t in range(N_ROUNDS):
        copies = []
        for g in range(ROUND // LANES):
            v = idx_v[pl.ds(t * ROUND + g * LANES, LANES)]
            for j in range(LANES):
                r = jnp.sum(jnp.where(lane_iota == j, v, 0))
                o = pl.multiple_of((r >> 3) * 8, 8)
                slot = g * LANES + j
                copies.append(
                    pltpu.async_copy(
                        table_hbm.at[pl.ds(o, 8), :],
                        band_v.at[pl.ds(slot * 8, 8), :],
                        sem,
                    )
                )
        for c in copies:
            c.wait()

        def compact(g, _, _t=t):
            v = idx_v[pl.ds((_t * ROUND) + g * LANES, LANES)]
            rows = (g * LANES + lane_iota) * 8 + (v & 7)
            cols = _t * ROUND + g * LANES + lane_iota
            for d in range(EMBED_DIM):
                got = plsc.load_gather(band_v, [rows, jnp.full((LANES,), d, jnp.int32)])
                plsc.store_scatter(
                    vals_v, [jnp.full((LANES,), d, jnp.int32), cols], got
                )
            return ()

        lax.fori_loop(0, ROUND // LANES, compact, (), unroll=False)

    pltpu.sync_copy(vals_v, out_hbm.at[:, pl.ds(base, B_PER_W)])


def kernel(item_idx, table):
    idx = item_idx.astype(jnp.int32).reshape(BATCH)
    return _gather_kernel(idx, table).T


# speedup vs baseline: 11.3806x; 1.5380x over previous
import functools

import jax
import jax.numpy as jnp
from jax import lax
from jax.experimental import pallas as pl
from jax.experimental.pallas import tpu as pltpu
from jax.experimental.pallas import tpu_sc as plsc

BATCH = 16384
EMBED_DIM = 32
NUM_CORES = 2
NUM_SUBCORES = 16
NUM_WORKERS = NUM_CORES * NUM_SUBCORES
B_PER_W = BATCH // NUM_WORKERS
ROUND = 64
N_ROUNDS = B_PER_W // ROUND
LANES = 16

_mesh = plsc.VectorSubcoreMesh(core_axis_name="c", subcore_axis_name="s")


@functools.partial(
    pl.kernel,
    out_type=jax.ShapeDtypeStruct((EMBED_DIM, BATCH), jnp.float32),
    mesh=_mesh,
    compiler_params=pltpu.CompilerParams(needs_layout_passes=False),
    scratch_types=[
        pltpu.VMEM((B_PER_W,), jnp.int32),
        pltpu.VMEM((ROUND * 8, EMBED_DIM), jnp.float32),
        pltpu.VMEM((EMBED_DIM, B_PER_W), jnp.float32),
        pltpu.SemaphoreType.DMA,
    ],
)
def _gather_kernel(idx_hbm, table_hbm, out_hbm, idx_v, band_v, vals_v, sem):
    wid = lax.axis_index("s") * NUM_CORES + lax.axis_index("c")
    base = wid * B_PER_W
    pltpu.sync_copy(idx_hbm.at[pl.ds(base, B_PER_W)], idx_v)

    lane_iota = lax.iota(jnp.int32, LANES)

    for t in range(N_ROUNDS):
        copies = []
        for g in range(ROUND // LANES):
            v = idx_v[pl.ds(t * ROUND + g * LANES, LANES)]
            for j in range(LANES):
                r = jnp.sum(jnp.where(lane_iota == j, v, 0))
                o = r >> 3
                slot = g * LANES + j
                copies.append(
                    pltpu.async_copy(
                        table_hbm.at[o],
                        band_v.at[pl.ds(slot * 8, 8), :],
                        sem,
                    )
                )
        for c in copies:
            c.wait()

        def compact(g, _, _t=t):
            v = idx_v[pl.ds((_t * ROUND) + g * LANES, LANES)]
            rows = (g * LANES + lane_iota) * 8 + (v & 7)
            cols = _t * ROUND + g * LANES + lane_iota
            for d in range(EMBED_DIM):
                got = plsc.load_gather(band_v, [rows, jnp.full((LANES,), d, jnp.int32)])
                plsc.store_scatter(
                    vals_v, [jnp.full((LANES,), d, jnp.int32), cols], got
                )
            return ()

        lax.fori_loop(0, ROUND // LANES, compact, (), unroll=False)

    pltpu.sync_copy(vals_v, out_hbm.at[:, pl.ds(base, B_PER_W)])


def kernel(item_idx, table):
    idx = item_idx.astype(jnp.int32).reshape(BATCH)
    t3 = table.reshape(BATCH and 125000, 8, EMBED_DIM)
    return _gather_kernel(idx, t3).T


# speedup vs baseline: 12.0580x; 1.0595x over previous
import functools

import jax
import jax.numpy as jnp
from jax import lax
from jax.experimental import pallas as pl
from jax.experimental.pallas import tpu as pltpu
from jax.experimental.pallas import tpu_sc as plsc

BATCH = 16384
EMBED_DIM = 32
NUM_ROWS = 1_000_000
NUM_BANDS = NUM_ROWS // 8
NUM_CORES = 2
NUM_SUBCORES = 16
NUM_WORKERS = NUM_CORES * NUM_SUBCORES
B_PER_W = BATCH // NUM_WORKERS
ROUND = 32
N_ROUNDS = B_PER_W // ROUND
LANES = 16

_mesh = plsc.VectorSubcoreMesh(core_axis_name="c", subcore_axis_name="s")


@functools.partial(
    pl.kernel,
    out_type=jax.ShapeDtypeStruct((EMBED_DIM, BATCH), jnp.float32),
    mesh=_mesh,
    compiler_params=pltpu.CompilerParams(needs_layout_passes=False),
    scratch_types=[
        pltpu.VMEM((B_PER_W,), jnp.int32),
        pltpu.VMEM((ROUND * 8, EMBED_DIM), jnp.float32),
        pltpu.VMEM((ROUND * 8, EMBED_DIM), jnp.float32),
        pltpu.VMEM((EMBED_DIM, B_PER_W), jnp.float32),
        pltpu.SemaphoreType.DMA,
        pltpu.SemaphoreType.DMA,
    ],
)
def _gather_kernel(idx_hbm, table_hbm, out_hbm, idx_v, band_a, band_b, vals_v,
                   sem_a, sem_b):
    wid = lax.axis_index("s") * NUM_CORES + lax.axis_index("c")
    base = wid * B_PER_W
    pltpu.sync_copy(idx_hbm.at[pl.ds(base, B_PER_W)], idx_v)

    lane_iota = lax.iota(jnp.int32, LANES)
    bufs = (band_a, band_b)
    sems = (sem_a, sem_b)
    pending = [None, None]

    def fire(t):
        buf, sem = bufs[t % 2], sems[t % 2]
        copies = []
        for g in range(ROUND // LANES):
            v = idx_v[pl.ds(t * ROUND + g * LANES, LANES)]
            for j in range(LANES):
                r = jnp.sum(jnp.where(lane_iota == j, v, 0))
                slot = g * LANES + j
                copies.append(
                    pltpu.async_copy(
                        table_hbm.at[r >> 3],
                        buf.at[pl.ds(slot * 8, 8), :],
                        sem,
                    )
                )
        pending[t % 2] = copies

    def drain_and_compact(t):
        for c in pending[t % 2]:
            c.wait()
        buf = bufs[t % 2]

        def compact(g, _, _t=t, _buf=buf):
            v = idx_v[pl.ds(_t * ROUND + g * LANES, LANES)]
            rows = (g * LANES + lane_iota) * 8 + (v & 7)
            cols = _t * ROUND + g * LANES + lane_iota
            for d in range(EMBED_DIM):
                dsplat = jnp.full((LANES,), d, jnp.int32)
                got = plsc.load_gather(_buf, [rows, dsplat])
                plsc.store_scatter(vals_v, [dsplat, cols], got)
            return ()

        lax.fori_loop(0, ROUND // LANES, compact, (), unroll=False)

    fire(0)
    for t in range(N_ROUNDS):
        if t + 1 < N_ROUNDS:
            fire(t + 1)
        drain_and_compact(t)

    pltpu.sync_copy(vals_v, out_hbm.at[:, pl.ds(base, B_PER_W)])


def kernel(item_idx, table):
    idx = item_idx.astype(jnp.int32).reshape(BATCH)
    t3 = table.reshape(NUM_BANDS, 8, EMBED_DIM)
    return _gather_kernel(idx, t3).T
